# baseline (device time: 77336 ns/iter reference)
import jax
import jax.numpy as jnp
from jax import lax
from jax.experimental import pallas as pl
from jax.experimental.pallas import tpu as pltpu

NC = 16
BAND_Y = range(0, 6)
BAND_X = range(6, 11)
BAND_Z = range(11, 16)
NY = len(BAND_Y)

ADD_ORDER = []
for _j in range(max(len(BAND_Y), len(BAND_X), len(BAND_Z))):
    for _band in (BAND_Y, BAND_X, BAND_Z):
        if _j < len(_band):
            ADD_ORDER.append(_band.start + _j)
assert sorted(ADD_ORDER) == list(range(NC))


def kernel(x):
    _, m, n2 = x.shape
    n = n2 // 2
    qw = n // 4
    rows = m // NC

    def body(x_hbm, out_ref, fq, ybuf, local_sem, stage_sem,
             y_send, y_recv, xa_send, xa_recv, za_send, za_recv,
             xr_send, xr_recv, zr_send, zr_recv):
        my_x = lax.axis_index("x")
        my_y = lax.axis_index("y")
        my_z = lax.axis_index("z")
        zp = my_z % 2
        y_peer = (my_x, 1 - my_y, my_z)
        x_peer = (1 - my_x, my_y, my_z)
        z_peer = (my_x, my_y, my_z + 1 - 2 * zp)

        qc_me = (2 * zp + my_x) * qw
        qc_x = (2 * zp + (1 - my_x)) * qw
        qc_z = (2 * (1 - zp) + my_x) * qw
        qc_diag = (2 * (1 - zp) + (1 - my_x)) * qw

        def mk(src, dst, ssem, rsem, dev):
            return pltpu.make_async_remote_copy(
                src_ref=src, dst_ref=dst, send_sem=ssem, recv_sem=rsem,
                device_id=dev, device_id_type=pl.DeviceIdType.MESH,
            )

        local = pltpu.make_async_copy(
            x_hbm.at[0, :, pl.ds(my_y * n, n)], out_ref, local_sem
        )
        local.start()

        fcol = (1 - my_y) * n

        stage_own = pltpu.make_async_copy(
            x_hbm.at[0, :, pl.ds(fcol + qc_me, qw)],
            ybuf.at[pl.ds(0, m)], stage_sem.at[0],
        )
        stage_own.start()
        diag_rows = NY * rows
        stage_diag = pltpu.make_async_copy(
            x_hbm.at[0, pl.ds(0, diag_rows), pl.ds(fcol + qc_diag, qw)],
            ybuf.at[pl.ds(m, diag_rows)], stage_sem.at[1],
        )
        stage_diag.start()

        barrier_sem = pltpu.get_barrier_semaphore()
        for peer in (y_peer, x_peer, z_peer):
            pl.semaphore_signal(
                barrier_sem, inc=1, device_id=peer,
                device_id_type=pl.DeviceIdType.MESH,
            )
        pl.semaphore_wait(barrier_sem, 3)

        y_out = []
        stage_own.wait()
        for k in range(NC):
            r = pl.ds(k * rows, rows)
            rd = mk(ybuf.at[r],
                    fq.at[r, pl.ds(qc_me, qw)],
                    y_send.at[k], y_recv.at[k], y_peer)
            rd.start()
            y_out.append(rd)
        y_diag = []
        stage_diag.wait()
        for j, k in enumerate(BAND_Y):
            r = pl.ds(k * rows, rows)
            rd = mk(ybuf.at[pl.ds(m + j * rows, rows)],
                    fq.at[r, pl.ds(qc_diag, qw)],
                    y_send.at[NC + j], y_recv.at[NC + j], y_peer)
            rd.start()
            y_diag.append(rd)

        xa_out, za_out = [], []
        for k in range(NC):
            r = pl.ds(k * rows, rows)
            y_out[k].wait_recv()
            rd = mk(fq.at[r, pl.ds(qc_me, qw)], fq.at[r, pl.ds(qc_me, qw)],
                    xa_send.at[k], xa_recv.at[k], x_peer)
            rd.start()
            xa_out.append(rd)
            rd = mk(fq.at[r, pl.ds(qc_me, qw)], fq.at[r, pl.ds(qc_me, qw)],
                    za_send.at[k], za_recv.at[k], z_peer)
            rd.start()
            za_out.append(rd)

        xa_in = [mk(fq.at[pl.ds(k * rows, rows), pl.ds(qc_x, qw)],
                    fq.at[pl.ds(k * rows, rows), pl.ds(qc_x, qw)],
                    xa_send.at[k], xa_recv.at[k], x_peer)
                 for k in range(NC)]
        za_in = [mk(fq.at[pl.ds(k * rows, rows), pl.ds(qc_z, qw)],
                    fq.at[pl.ds(k * rows, rows), pl.ds(qc_z, qw)],
                    za_send.at[k], za_recv.at[k], z_peer)
                 for k in range(NC)]

        xr_out, zr_out = [], []
        for k in range(NC):
            r = pl.ds(k * rows, rows)
            za_in[k].wait_recv()
            if k in BAND_X:
                j = k - BAND_X.start
                rd = mk(fq.at[r, pl.ds(qc_z, qw)], fq.at[r, pl.ds(qc_z, qw)],
                        xr_send.at[j], xr_recv.at[j], x_peer)
                rd.start()
                xr_out.append(rd)
            xa_in[k].wait_recv()
            if k in BAND_Z:
                j = k - BAND_Z.start
                rd = mk(fq.at[r, pl.ds(qc_x, qw)], fq.at[r, pl.ds(qc_x, qw)],
                        zr_send.at[j], zr_recv.at[j], z_peer)
                rd.start()
                zr_out.append(rd)

        local.wait()
        for k in ADD_ORDER:
            r = pl.ds(k * rows, rows)
            if k in BAND_Y:
                y_diag[k - BAND_Y.start].wait_recv()
            elif k in BAND_X:
                j = k - BAND_X.start
                mk(fq.at[r, pl.ds(qc_diag, qw)], fq.at[r, pl.ds(qc_diag, qw)],
                   xr_send.at[j], xr_recv.at[j], x_peer).wait_recv()
            else:
                j = k - BAND_Z.start
                mk(fq.at[r, pl.ds(qc_diag, qw)], fq.at[r, pl.ds(qc_diag, qw)],
                   zr_send.at[j], zr_recv.at[j], z_peer).wait_recv()
            out_ref[r, :] += fq[r, :]

        for rd in y_out + y_diag + xa_out + za_out + xr_out + zr_out:
            rd.wait_send()

    return pl.pallas_call(
        body,
        out_shape=jax.ShapeDtypeStruct((m, n), jnp.float32),
        in_specs=[pl.BlockSpec(memory_space=pl.ANY)],
        out_specs=pl.BlockSpec(memory_space=pltpu.VMEM),
        scratch_shapes=[
            pltpu.VMEM((m, n), jnp.float32),
            pltpu.VMEM((m + (m // NC) * NY, n // 4), jnp.float32),
            pltpu.SemaphoreType.DMA,
            pltpu.SemaphoreType.DMA((2,)),
            pltpu.SemaphoreType.DMA((NC + NY,)),
            pltpu.SemaphoreType.DMA((NC + NY,)),
            pltpu.SemaphoreType.DMA((NC,)),
            pltpu.SemaphoreType.DMA((NC,)),
            pltpu.SemaphoreType.DMA((NC,)),
            pltpu.SemaphoreType.DMA((NC,)),
            pltpu.SemaphoreType.DMA((len(BAND_X),)),
            pltpu.SemaphoreType.DMA((len(BAND_X),)),
            pltpu.SemaphoreType.DMA((len(BAND_Z),)),
            pltpu.SemaphoreType.DMA((len(BAND_Z),)),
        ],
        compiler_params=pltpu.CompilerParams(collective_id=0),
    )(x)


# device time: 76841 ns/iter; 1.0064x vs baseline; 1.0064x over previous
import jax
import jax.numpy as jnp
from jax import lax
from jax.experimental import pallas as pl
from jax.experimental.pallas import tpu as pltpu

NC = 16
BAND_Y = range(0, 6)
BAND_X = range(6, 11)
BAND_Z = range(11, 16)
NY = len(BAND_Y)

ADD_ORDER = []
for _j in range(max(len(BAND_Y), len(BAND_X), len(BAND_Z))):
    for _band in (BAND_Y, BAND_X, BAND_Z):
        if _j < len(_band):
            ADD_ORDER.append(_band.start + _j)
assert sorted(ADD_ORDER) == list(range(NC))


def kernel(x):
    _, m, n2 = x.shape
    n = n2 // 2
    qw = n // 4
    rows = m // NC

    def body(x_hbm, out_ref, fq, local_sem,
             y_send, y_recv, xa_send, xa_recv, za_send, za_recv,
             xr_send, xr_recv, zr_send, zr_recv):
        my_x = lax.axis_index("x")
        my_y = lax.axis_index("y")
        my_z = lax.axis_index("z")
        zp = my_z % 2
        y_peer = (my_x, 1 - my_y, my_z)
        x_peer = (1 - my_x, my_y, my_z)
        z_peer = (my_x, my_y, my_z + 1 - 2 * zp)

        qc_me = (2 * zp + my_x) * qw
        qc_x = (2 * zp + (1 - my_x)) * qw
        qc_z = (2 * (1 - zp) + my_x) * qw
        qc_diag = (2 * (1 - zp) + (1 - my_x)) * qw

        def mk(src, dst, ssem, rsem, dev):
            return pltpu.make_async_remote_copy(
                src_ref=src, dst_ref=dst, send_sem=ssem, recv_sem=rsem,
                device_id=dev, device_id_type=pl.DeviceIdType.MESH,
            )

        local = pltpu.make_async_copy(
            x_hbm.at[0, :, pl.ds(my_y * n, n)], out_ref, local_sem
        )
        local.start()

        barrier_sem = pltpu.get_barrier_semaphore()
        for peer in (y_peer, x_peer, z_peer):
            pl.semaphore_signal(
                barrier_sem, inc=1, device_id=peer,
                device_id_type=pl.DeviceIdType.MESH,
            )
        pl.semaphore_wait(barrier_sem, 3)

        fcol = (1 - my_y) * n

        y_out = []
        for k in range(NC):
            r = pl.ds(k * rows, rows)
            rd = mk(x_hbm.at[0, r, pl.ds(fcol + qc_me, qw)],
                    fq.at[r, pl.ds(qc_me, qw)],
                    y_send.at[k], y_recv.at[k], y_peer)
            rd.start()
            y_out.append(rd)
        y_diag = []
        for j, k in enumerate(BAND_Y):
            r = pl.ds(k * rows, rows)
            rd = mk(x_hbm.at[0, r, pl.ds(fcol + qc_diag, qw)],
                    fq.at[r, pl.ds(qc_diag, qw)],
                    y_send.at[NC + j], y_recv.at[NC + j], y_peer)
            rd.start()
            y_diag.append(rd)

        xa_out, za_out = [], []
        for k in range(NC):
            r = pl.ds(k * rows, rows)
            y_out[k].wait_recv()
            rd = mk(fq.at[r, pl.ds(qc_me, qw)], fq.at[r, pl.ds(qc_me, qw)],
                    xa_send.at[k], xa_recv.at[k], x_peer)
            rd.start()
            xa_out.append(rd)
            rd = mk(fq.at[r, pl.ds(qc_me, qw)], fq.at[r, pl.ds(qc_me, qw)],
                    za_send.at[k], za_recv.at[k], z_peer)
            rd.start()
            za_out.append(rd)

        xa_in = [mk(fq.at[pl.ds(k * rows, rows), pl.ds(qc_x, qw)],
                    fq.at[pl.ds(k * rows, rows), pl.ds(qc_x, qw)],
                    xa_send.at[k], xa_recv.at[k], x_peer)
                 for k in range(NC)]
        za_in = [mk(fq.at[pl.ds(k * rows, rows), pl.ds(qc_z, qw)],
                    fq.at[pl.ds(k * rows, rows), pl.ds(qc_z, qw)],
                    za_send.at[k], za_recv.at[k], z_peer)
                 for k in range(NC)]

        xr_out, zr_out = [], []
        for k in range(NC):
            r = pl.ds(k * rows, rows)
            za_in[k].wait_recv()
            if k in BAND_X:
                j = k - BAND_X.start
                rd = mk(fq.at[r, pl.ds(qc_z, qw)], fq.at[r, pl.ds(qc_z, qw)],
                        xr_send.at[j], xr_recv.at[j], x_peer)
                rd.start()
                xr_out.append(rd)
            xa_in[k].wait_recv()
            if k in BAND_Z:
                j = k - BAND_Z.start
                rd = mk(fq.at[r, pl.ds(qc_x, qw)], fq.at[r, pl.ds(qc_x, qw)],
                        zr_send.at[j], zr_recv.at[j], z_peer)
                rd.start()
                zr_out.append(rd)

        local.wait()
        for k in ADD_ORDER:
            r = pl.ds(k * rows, rows)
            if k in BAND_Y:
                y_diag[k - BAND_Y.start].wait_recv()
            elif k in BAND_X:
                j = k - BAND_X.start
                mk(fq.at[r, pl.ds(qc_diag, qw)], fq.at[r, pl.ds(qc_diag, qw)],
                   xr_send.at[j], xr_recv.at[j], x_peer).wait_recv()
            else:
                j = k - BAND_Z.start
                mk(fq.at[r, pl.ds(qc_diag, qw)], fq.at[r, pl.ds(qc_diag, qw)],
                   zr_send.at[j], zr_recv.at[j], z_peer).wait_recv()
            out_ref[r, :] += fq[r, :]

        for rd in y_out + y_diag + xa_out + za_out + xr_out + zr_out:
            rd.wait_send()

    return pl.pallas_call(
        body,
        out_shape=jax.ShapeDtypeStruct((m, n), jnp.float32),
        in_specs=[pl.BlockSpec(memory_space=pl.ANY)],
        out_specs=pl.BlockSpec(memory_space=pltpu.VMEM),
        scratch_shapes=[
            pltpu.VMEM((m, n), jnp.float32),
            pltpu.SemaphoreType.DMA,
            pltpu.SemaphoreType.DMA((NC + NY,)),
            pltpu.SemaphoreType.DMA((NC + NY,)),
            pltpu.SemaphoreType.DMA((NC,)),
            pltpu.SemaphoreType.DMA((NC,)),
            pltpu.SemaphoreType.DMA((NC,)),
            pltpu.SemaphoreType.DMA((NC,)),
            pltpu.SemaphoreType.DMA((len(BAND_X),)),
            pltpu.SemaphoreType.DMA((len(BAND_X),)),
            pltpu.SemaphoreType.DMA((len(BAND_Z),)),
            pltpu.SemaphoreType.DMA((len(BAND_Z),)),
        ],
        compiler_params=pltpu.CompilerParams(collective_id=0),
    )(x)


# device time: 76490 ns/iter; 1.0111x vs baseline; 1.0046x over previous
import jax
import jax.numpy as jnp
from jax import lax
from jax.experimental import pallas as pl
from jax.experimental.pallas import tpu as pltpu

NC = 32
BAND_Y = range(0, 11)
BAND_X = range(11, 22)
BAND_Z = range(22, 32)
NY = len(BAND_Y)

ADD_ORDER = []
for _j in range(max(len(BAND_Y), len(BAND_X), len(BAND_Z))):
    for _band in (BAND_Y, BAND_X, BAND_Z):
        if _j < len(_band):
            ADD_ORDER.append(_band.start + _j)
assert sorted(ADD_ORDER) == list(range(NC))


def kernel(x):
    _, m, n2 = x.shape
    n = n2 // 2
    qw = n // 4
    rows = m // NC

    def body(x_hbm, out_ref, fq, local_sem,
             y_send, y_recv, xa_send, xa_recv, za_send, za_recv,
             xr_send, xr_recv, zr_send, zr_recv):
        my_x = lax.axis_index("x")
        my_y = lax.axis_index("y")
        my_z = lax.axis_index("z")
        zp = my_z % 2
        y_peer = (my_x, 1 - my_y, my_z)
        x_peer = (1 - my_x, my_y, my_z)
        z_peer = (my_x, my_y, my_z + 1 - 2 * zp)

        qc_me = (2 * zp + my_x) * qw
        qc_x = (2 * zp + (1 - my_x)) * qw
        qc_z = (2 * (1 - zp) + my_x) * qw
        qc_diag = (2 * (1 - zp) + (1 - my_x)) * qw

        def mk(src, dst, ssem, rsem, dev):
            return pltpu.make_async_remote_copy(
                src_ref=src, dst_ref=dst, send_sem=ssem, recv_sem=rsem,
                device_id=dev, device_id_type=pl.DeviceIdType.MESH,
            )

        local = pltpu.make_async_copy(
            x_hbm.at[0, :, pl.ds(my_y * n, n)], out_ref, local_sem
        )
        local.start()

        barrier_sem = pltpu.get_barrier_semaphore()
        for peer in (y_peer, x_peer, z_peer):
            pl.semaphore_signal(
                barrier_sem, inc=1, device_id=peer,
                device_id_type=pl.DeviceIdType.MESH,
            )
        pl.semaphore_wait(barrier_sem, 3)

        fcol = (1 - my_y) * n

        y_out = []
        for k in range(NC):
            r = pl.ds(k * rows, rows)
            rd = mk(x_hbm.at[0, r, pl.ds(fcol + qc_me, qw)],
                    fq.at[r, pl.ds(qc_me, qw)],
                    y_send.at[k], y_recv.at[k], y_peer)
            rd.start()
            y_out.append(rd)
        y_diag = []
        for j, k in enumerate(BAND_Y):
            r = pl.ds(k * rows, rows)
            rd = mk(x_hbm.at[0, r, pl.ds(fcol + qc_diag, qw)],
                    fq.at[r, pl.ds(qc_diag, qw)],
                    y_send.at[NC + j], y_recv.at[NC + j], y_peer)
            rd.start()
            y_diag.append(rd)

        xa_out, za_out = [], []
        for k in range(NC):
            r = pl.ds(k * rows, rows)
            y_out[k].wait_recv()
            rd = mk(fq.at[r, pl.ds(qc_me, qw)], fq.at[r, pl.ds(qc_me, qw)],
                    xa_send.at[k], xa_recv.at[k], x_peer)
            rd.start()
            xa_out.append(rd)
            rd = mk(fq.at[r, pl.ds(qc_me, qw)], fq.at[r, pl.ds(qc_me, qw)],
                    za_send.at[k], za_recv.at[k], z_peer)
            rd.start()
            za_out.append(rd)

        xa_in = [mk(fq.at[pl.ds(k * rows, rows), pl.ds(qc_x, qw)],
                    fq.at[pl.ds(k * rows, rows), pl.ds(qc_x, qw)],
                    xa_send.at[k], xa_recv.at[k], x_peer)
                 for k in range(NC)]
        za_in = [mk(fq.at[pl.ds(k * rows, rows), pl.ds(qc_z, qw)],
                    fq.at[pl.ds(k * rows, rows), pl.ds(qc_z, qw)],
                    za_send.at[k], za_recv.at[k], z_peer)
                 for k in range(NC)]

        xr_out, zr_out = [], []
        for k in range(NC):
            r = pl.ds(k * rows, rows)
            za_in[k].wait_recv()
            if k in BAND_X:
                j = k - BAND_X.start
                rd = mk(fq.at[r, pl.ds(qc_z, qw)], fq.at[r, pl.ds(qc_z, qw)],
                        xr_send.at[j], xr_recv.at[j], x_peer)
                rd.start()
                xr_out.append(rd)
            xa_in[k].wait_recv()
            if k in BAND_Z:
                j = k - BAND_Z.start
                rd = mk(fq.at[r, pl.ds(qc_x, qw)], fq.at[r, pl.ds(qc_x, qw)],
                        zr_send.at[j], zr_recv.at[j], z_peer)
                rd.start()
                zr_out.append(rd)

        local.wait()
        for k in ADD_ORDER:
            r = pl.ds(k * rows, rows)
            if k in BAND_Y:
                y_diag[k - BAND_Y.start].wait_recv()
            elif k in BAND_X:
                j = k - BAND_X.start
                mk(fq.at[r, pl.ds(qc_diag, qw)], fq.at[r, pl.ds(qc_diag, qw)],
                   xr_send.at[j], xr_recv.at[j], x_peer).wait_recv()
            else:
                j = k - BAND_Z.start
                mk(fq.at[r, pl.ds(qc_diag, qw)], fq.at[r, pl.ds(qc_diag, qw)],
                   zr_send.at[j], zr_recv.at[j], z_peer).wait_recv()
            out_ref[r, :] += fq[r, :]

        for rd in y_out + y_diag + xa_out + za_out + xr_out + zr_out:
            rd.wait_send()

    return pl.pallas_call(
        body,
        out_shape=jax.ShapeDtypeStruct((m, n), jnp.float32),
        in_specs=[pl.BlockSpec(memory_space=pl.ANY)],
        out_specs=pl.BlockSpec(memory_space=pltpu.VMEM),
        scratch_shapes=[
            pltpu.VMEM((m, n), jnp.float32),
            pltpu.SemaphoreType.DMA,
            pltpu.SemaphoreType.DMA((NC + NY,)),
            pltpu.SemaphoreType.DMA((NC + NY,)),
            pltpu.SemaphoreType.DMA((NC,)),
            pltpu.SemaphoreType.DMA((NC,)),
            pltpu.SemaphoreType.DMA((NC,)),
            pltpu.SemaphoreType.DMA((NC,)),
            pltpu.SemaphoreType.DMA((len(BAND_X),)),
            pltpu.SemaphoreType.DMA((len(BAND_X),)),
            pltpu.SemaphoreType.DMA((len(BAND_Z),)),
            pltpu.SemaphoreType.DMA((len(BAND_Z),)),
        ],
        compiler_params=pltpu.CompilerParams(collective_id=0),
    )(x)
